# baseline (speedup 1.0000x reference)
import functools

import jax
import jax.numpy as jnp
from jax import lax
from jax.experimental import pallas as pl
from jax.experimental.pallas import tpu as pltpu
from jax.experimental.pallas import tpu_sc as plsc

NC = 2
NS = 16
LN = 16

H = 32
DW = 16

CHUNK = 128
RPB = 8


def _ceil_to(x, m):
    return (x + m - 1) // m * m


IB = RPB * CHUNK
SEGQ = 2 * IB


def _partition(row1d, col1d, ew1d, half, cap):
    ep = row1d.shape[0]
    ep32 = ep // (NC * NS)
    nblk = ep32 // IB

    mesh = plsc.VectorSubcoreMesh(core_axis_name="c", subcore_axis_name="s",
                                  num_cores=NC, num_subcores=NS)
    nreg = NC * NS

    @functools.partial(
        pl.kernel,
        out_type=[
            jax.ShapeDtypeStruct((2 * nreg * cap,), jnp.int32),
            jax.ShapeDtypeStruct((2 * nreg * cap,), jnp.int32),
            jax.ShapeDtypeStruct((2 * nreg * cap,), jnp.float32),
            jax.ShapeDtypeStruct((2 * nreg * 16,), jnp.int32),
        ],
        mesh=mesh,
        compiler_params=pltpu.CompilerParams(use_tc_tiling_on_sc=False,
                                             needs_layout_passes=False),
        scratch_types=[
            pltpu.VMEM((IB,), jnp.int32),
            pltpu.VMEM((IB,), jnp.int32),
            pltpu.VMEM((IB,), jnp.float32),
            pltpu.VMEM((160,), jnp.int32),
            pltpu.VMEM((160,), jnp.int32),
            pltpu.VMEM((160,), jnp.float32),
            pltpu.VMEM((160,), jnp.int32),
            pltpu.VMEM((160,), jnp.int32),
            pltpu.VMEM((160,), jnp.float32),
            pltpu.VMEM((CHUNK,), jnp.int32),
            pltpu.VMEM((CHUNK,), jnp.float32),
            pltpu.VMEM((16,), jnp.int32),
        ],
    )
    def kern(row_hbm, col_hbm, ew_hbm, rowp, colp, ewp, cnts,
             rowv, colv, ewv, sr0, sc0, se0, sr1, sc1, se1, zi, zf, cntv):
        cid = lax.axis_index("c")
        sid = lax.axis_index("s")
        w = sid * NC + cid
        in_base = w * ep32
        stg = ((sr0, sc0, se0), (sr1, sc1, se1))
        outs = (rowp, colp, ewp)

        @pl.loop(0, CHUNK, step=LN)
        def _(r):
            zi[pl.ds(r, LN)] = jnp.zeros((LN,), jnp.int32)
            zf[pl.ds(r, LN)] = jnp.zeros((LN,), jnp.float32)

        def flush(p, optr):
            ob = pl.multiple_of((p * nreg + w) * cap + optr, CHUNK)
            for st, out in zip(stg[p], outs):
                pltpu.sync_copy(st.at[pl.ds(0, CHUNK)],
                                out.at[pl.ds(ob, CHUNK)])

        def shift_residual(p):
            for st in stg[p]:
                res = st[pl.ds(CHUNK, LN)]
                st[pl.ds(0, LN)] = res

        def compact_group(vals, m, p, sptr, optr):
            cnt = plsc.all_reduce_population_count(m)[0]
            for st, v in zip(stg[p], vals):
                plsc.store_compressed(st.at[pl.ds(sptr, LN)], v, mask=m)
            sptr = sptr + cnt

            def do_flush(args):
                sptr, optr = args
                flush(p, optr)
                shift_residual(p)
                return sptr - CHUNK, optr + CHUNK

            return lax.cond(sptr >= CHUNK, do_flush, lambda a: a, (sptr, optr))

        def block_body(b, carry):
            s0, o0, s1, o1 = carry
            e0 = pl.multiple_of(in_base + b * IB, CHUNK)
            pltpu.sync_copy(row_hbm.at[pl.ds(e0, IB)], rowv)
            pltpu.sync_copy(col_hbm.at[pl.ds(e0, IB)], colv)
            pltpu.sync_copy(ew_hbm.at[pl.ds(e0, IB)], ewv)

            def group_body(q, carry):
                s0, o0, s1, o1 = carry
                r16 = rowv[pl.ds(q * LN, LN)]
                c16 = colv[pl.ds(q * LN, LN)]
                e16 = ewv[pl.ds(q * LN, LN)]
                m1 = c16 >= half
                s0, o0 = compact_group((r16, c16, e16), ~m1, 0, s0, o0)
                s1, o1 = compact_group((r16, c16 - half, e16), m1, 1, s1, o1)
                return s0, o0, s1, o1

            return lax.fori_loop(0, IB // LN, group_body, (s0, o0, s1, o1))

        s0, o0, s1, o1 = lax.fori_loop(0, nblk, block_body, (0, 0, 0, 0))

        def finish(p, sptr, optr):
            rpad = (LN - sptr % LN) % LN
            m = lax.iota(jnp.int32, LN) < rpad
            for st in stg[p]:
                plsc.store_compressed(st.at[pl.ds(sptr, LN)],
                                      jnp.zeros((LN,), st.dtype), mask=m)
            sptr = sptr + rpad

            d16 = ((CHUNK - sptr % CHUNK) % CHUNK) // LN

            def pad_body(i, sp):
                for st in stg[p]:
                    st[pl.ds(sp, LN)] = jnp.zeros((LN,), st.dtype)
                return sp + LN

            sptr = lax.fori_loop(0, d16, pad_body, sptr)

            @pl.when(sptr > 0)
            def _():
                flush(p, optr)

            optr = optr + sptr
            target = jnp.maximum((optr + SEGQ - 1) // SEGQ * SEGQ, SEGQ)
            kd = (target - optr) // CHUNK

            def dummy_body(i, o):
                ob = pl.multiple_of((p * nreg + w) * cap + o, CHUNK)
                pltpu.sync_copy(zi, rowp.at[pl.ds(ob, CHUNK)])
                pltpu.sync_copy(zi, colp.at[pl.ds(ob, CHUNK)])
                pltpu.sync_copy(zf, ewp.at[pl.ds(ob, CHUNK)])
                return o + CHUNK

            lax.fori_loop(0, kd, dummy_body, optr)
            cntv[...] = jnp.full((LN,), target, jnp.int32)
            cb = pl.multiple_of((p * nreg + w) * 16, 16)
            pltpu.sync_copy(cntv, cnts.at[pl.ds(cb, 16)])

        finish(0, s0, o0)
        finish(1, s1, o1)

    return kern(row1d, col1d, ew1d)


def _seg_sum(hp, rowp, colp, ewp, counts, n, tot, half, cap):
    wb_a = _ceil_to(-(-half // NS), 8)
    wb_last = half - (NS - 1) * wb_a
    nreg = NC * NS

    mesh = plsc.VectorSubcoreMesh(core_axis_name="c", subcore_axis_name="s",
                                  num_cores=NC, num_subcores=NS)

    @functools.partial(
        pl.kernel,
        out_type=jax.ShapeDtypeStruct((n, H), jnp.float32),
        mesh=mesh,
        compiler_params=pltpu.CompilerParams(use_tc_tiling_on_sc=False),
        scratch_types=[
            pltpu.VMEM((2, RPB * CHUNK), jnp.int32),
            pltpu.VMEM((2, RPB * CHUNK), jnp.int32),
            pltpu.VMEM((2, RPB * CHUNK), jnp.float32),
            pltpu.VMEM((2, CHUNK), jnp.int32),
            pltpu.VMEM((2, CHUNK, H), jnp.float32),
            pltpu.VMEM((2, CHUNK, H), jnp.float32),
            pltpu.VMEM((64, H), jnp.float32),
            pltpu.VMEM((16,), jnp.int32),
            pltpu.VMEM_SHARED((tot, H), jnp.float32),
            pltpu.SemaphoreType.DMA,
            pltpu.SemaphoreType.DMA,
            pltpu.SemaphoreType.DMA,
        ],
    )
    def kern(hp_hbm, row_hbm, col_hbm, ew_hbm, cnt_hbm, out_hbm,
             rowv, colv, ewv, lidxv, gathv, msgv, zv, cntv, accs,
             sem_i, sem_g, sem_s):
        cid = lax.axis_index("c")
        sid = lax.axis_index("s")
        base = cid * half
        ib = RPB * CHUNK

        @pl.loop(0, 64)
        def _(r):
            @pl.loop(0, H, step=LN)
            def _(hh):
                zv.at[r][pl.ds(hh, LN)] = jnp.zeros((LN,), jnp.float32)

        zslice = tot // NS

        @pl.loop(0, zslice, step=64)
        def _(r):
            pltpu.sync_copy(zv, accs.at[pl.ds(sid * zslice + r, 64)])

        plsc.subcore_barrier()

        def idx_issue(ebase, b, pi):
            e0 = pl.multiple_of(ebase + b * ib, CHUNK)
            pltpu.async_copy(row_hbm.at[pl.ds(e0, ib)], rowv.at[pi], sem_i)
            pltpu.async_copy(col_hbm.at[pl.ds(e0, ib)], colv.at[pi], sem_i)
            pltpu.async_copy(ew_hbm.at[pl.ds(e0, ib)], ewv.at[pi], sem_i)

        def idx_drain(pi):
            pltpu.make_async_copy(row_hbm.at[pl.ds(0, ib)], rowv.at[pi], sem_i).wait()
            pltpu.make_async_copy(col_hbm.at[pl.ds(0, ib)], colv.at[pi], sem_i).wait()
            pltpu.make_async_copy(ew_hbm.at[pl.ds(0, ib)], ewv.at[pi], sem_i).wait()

        def gat_issue(c, pi):
            pltpu.async_copy(
                hp_hbm.at[rowv.at[pi].at[pl.ds(c * CHUNK, CHUNK)]],
                gathv.at[c % 2], sem_g)

        def gat_drain(c, pi):
            pltpu.make_async_copy(
                hp_hbm.at[rowv.at[pi].at[pl.ds(c * CHUNK, CHUNK)]],
                gathv.at[c % 2], sem_g).wait()

        def sc_issue(sp):
            pltpu.async_copy(msgv.at[sp], accs.at[lidxv.at[sp]], sem_s, add=True)

        def sc_drain(sp):
            pltpu.make_async_copy(msgv.at[sp], accs.at[lidxv.at[sp]], sem_s).wait()

        def do_chunk(bb, c, sp, pi):
            gat_drain(c, pi)

            @pl.when((bb > 0) | (c >= 2))
            def _():
                sc_drain(sp)

            for q in range(CHUNK // LN):
                lidxv.at[sp][pl.ds(q * LN, LN)] = (
                    colv.at[pi][pl.ds(c * CHUNK + q * LN, LN)])
                ew16 = ewv.at[pi][pl.ds(c * CHUNK + q * LN, LN)]
                for j in range(LN):
                    e = q * LN + j
                    w = jnp.full((LN,), ew16[j], jnp.float32)
                    mrow = msgv.at[sp].at[e]
                    grow = gathv.at[sp].at[e]
                    for hh in range(H // LN):
                        sl = pl.ds(hh * LN, LN)
                        mrow[sl] = grow[sl] * w

            sc_issue(sp)

        def do_block(ebase, nblk_r, bb, pi):
            idx_drain(pi)

            @pl.when(bb + 1 < nblk_r)
            def _():
                idx_issue(ebase, bb + 1, 1 - pi)

            gat_issue(0, pi)

            @pl.loop(0, RPB, step=2)
            def _(c):
                for off in (0, 1):
                    cc = c + off

                    @pl.when(cc + 1 < RPB)
                    def _():
                        gat_issue(cc + 1, pi)

                    do_chunk(bb, cc, off, pi)

        for k in (0, 1):
            reg = 2 * sid + k
            slot = cid * nreg + reg
            ebase = slot * cap
            pltpu.sync_copy(cnt_hbm.at[pl.ds(pl.multiple_of(slot * 16, 16), 16)],
                            cntv)
            nblk_r = cntv[...][0] // ib

            idx_issue(ebase, 0, 0)

            @pl.loop(0, nblk_r, step=2)
            def _(b):
                do_block(ebase, nblk_r, b, 0)
                do_block(ebase, nblk_r, b + 1, 1)

            sc_drain(0)
            sc_drain(1)

        plsc.subcore_barrier()

        @pl.when(sid < NS - 1)
        def _():
            pltpu.sync_copy(accs.at[pl.ds(sid * wb_a, wb_a)],
                            out_hbm.at[pl.ds(base + sid * wb_a, wb_a)])

        @pl.when(sid == NS - 1)
        def _():
            pltpu.sync_copy(accs.at[pl.ds((NS - 1) * wb_a, wb_last)],
                            out_hbm.at[pl.ds(base + (NS - 1) * wb_a, wb_last)])

    return kern(hp, rowp, colp, ewp, counts)


def _deg_sum(col1d, ew1d, n, tot, half):
    ep = col1d.shape[0]
    per_sub = ep // NS
    nblk = per_sub // (RPB * CHUNK)
    wb_a = _ceil_to(-(-half // NS), 8)
    wb_last = half - (NS - 1) * wb_a

    mesh = plsc.VectorSubcoreMesh(core_axis_name="c", subcore_axis_name="s",
                                  num_cores=NC, num_subcores=NS)

    @functools.partial(
        pl.kernel,
        out_type=jax.ShapeDtypeStruct((n, DW), jnp.float32),
        mesh=mesh,
        compiler_params=pltpu.CompilerParams(use_tc_tiling_on_sc=False),
        scratch_types=[
            pltpu.VMEM((2, RPB * CHUNK), jnp.int32),
            pltpu.VMEM((2, RPB * CHUNK), jnp.float32),
            pltpu.VMEM((2, CHUNK), jnp.int32),
            pltpu.VMEM((2, CHUNK, DW), jnp.float32),
            pltpu.VMEM((64, DW), jnp.float32),
            pltpu.VMEM_SHARED((tot, DW), jnp.float32),
            pltpu.SemaphoreType.DMA,
            pltpu.SemaphoreType.DMA,
        ],
    )
    def kern(col_hbm, ew_hbm, out_hbm, colv, ewv, lidxv, msgv, zv, accs,
             sem_i, sem_s):
        cid = lax.axis_index("c")
        sid = lax.axis_index("s")
        base = cid * half
        ib = RPB * CHUNK

        @pl.loop(0, 64)
        def _(r):
            zv.at[r][...] = jnp.zeros((LN,), jnp.float32)

        zslice = tot // NS

        @pl.loop(0, zslice, step=64)
        def _(r):
            pltpu.sync_copy(zv, accs.at[pl.ds(sid * zslice + r, 64)])

        plsc.subcore_barrier()

        def idx_issue(b, pi):
            e0 = sid * per_sub + b * ib
            pltpu.async_copy(col_hbm.at[pl.ds(e0, ib)], colv.at[pi], sem_i)
            pltpu.async_copy(ew_hbm.at[pl.ds(e0, ib)], ewv.at[pi], sem_i)

        def idx_drain(pi):
            pltpu.make_async_copy(col_hbm.at[pl.ds(0, ib)], colv.at[pi], sem_i).wait()
            pltpu.make_async_copy(ew_hbm.at[pl.ds(0, ib)], ewv.at[pi], sem_i).wait()

        def sc_issue(sp):
            pltpu.async_copy(msgv.at[sp], accs.at[lidxv.at[sp]], sem_s, add=True)

        def sc_drain(sp):
            pltpu.make_async_copy(msgv.at[sp], accs.at[lidxv.at[sp]], sem_s).wait()

        def do_chunk(bb, c, sp, pi):
            @pl.when((bb > 0) | (c >= 2))
            def _():
                sc_drain(sp)

            for q in range(CHUNK // LN):
                cols = colv.at[pi][pl.ds(c * CHUNK + q * LN, LN)]
                loc = cols - base
                oob = (loc < 0) | (loc >= half)
                trash = half + sid * 8 + (q % 8)
                loc = jnp.where(oob, jnp.full((LN,), trash, jnp.int32), loc)
                lidxv.at[sp][pl.ds(q * LN, LN)] = loc
                ew16 = ewv.at[pi][pl.ds(c * CHUNK + q * LN, LN)]
                for j in range(LN):
                    e = q * LN + j
                    msgv.at[sp].at[e][...] = jnp.full((LN,), ew16[j], jnp.float32)

            sc_issue(sp)

        def do_block(bb, pi):
            idx_drain(pi)

            @pl.when(bb + 1 < nblk)
            def _():
                idx_issue(bb + 1, 1 - pi)

            @pl.loop(0, RPB, step=2)
            def _(c):
                for off in (0, 1):
                    do_chunk(bb, c + off, off, pi)

        idx_issue(0, 0)

        @pl.loop(0, nblk, step=2)
        def _(b):
            do_block(b, 0)
            do_block(b + 1, 1)

        sc_drain(0)
        sc_drain(1)

        plsc.subcore_barrier()

        @pl.when(sid < NS - 1)
        def _():
            pltpu.sync_copy(accs.at[pl.ds(sid * wb_a, wb_a)],
                            out_hbm.at[pl.ds(base + sid * wb_a, wb_a)])

        @pl.when(sid == NS - 1)
        def _():
            pltpu.sync_copy(accs.at[pl.ds((NS - 1) * wb_a, wb_last)],
                            out_hbm.at[pl.ds(base + (NS - 1) * wb_a, wb_last)])

    return kern(col1d, ew1d)


BLK = 2000


def _elu(x):
    return jnp.where(x > 0, x, jnp.exp(x) - 1.0)


def _dinv_of(degacc):
    deg = degacc[:, :1] + 1.0
    return jnp.where(deg > 0, lax.rsqrt(jnp.where(deg > 0, deg, 1.0)), 0.0)


def _tc_mm(x, W1, Wf, bf, W2):
    n, f_in = x.shape

    def body(x_ref, w1_ref, wf_ref, bf_ref, w2_ref, g1_ref, g2_ref):
        xb = x_ref[...]
        g1_ref[...] = jnp.dot(xb, w1_ref[...],
                              preferred_element_type=jnp.float32)
        h0 = _elu(jnp.dot(xb, wf_ref[...],
                          preferred_element_type=jnp.float32) + bf_ref[...])
        g2_ref[...] = jnp.dot(h0, w2_ref[...],
                              preferred_element_type=jnp.float32)

    return pl.pallas_call(
        body,
        grid=(n // BLK,),
        in_specs=[
            pl.BlockSpec((BLK, f_in), lambda i: (i, 0)),
            pl.BlockSpec((f_in, H), lambda i: (0, 0)),
            pl.BlockSpec((f_in, H), lambda i: (0, 0)),
            pl.BlockSpec((1, H), lambda i: (0, 0)),
            pl.BlockSpec((H, H), lambda i: (0, 0)),
        ],
        out_specs=[
            pl.BlockSpec((BLK, H), lambda i: (i, 0)),
            pl.BlockSpec((BLK, H), lambda i: (i, 0)),
        ],
        out_shape=[
            jax.ShapeDtypeStruct((n, H), jnp.float32),
            jax.ShapeDtypeStruct((n, H), jnp.float32),
        ],
    )(x, W1, Wf, bf, W2)


def _tc_scale(g1, g2, degacc):
    n = g1.shape[0]

    def body(g1_ref, g2_ref, d_ref, hp1_ref, hp2_ref):
        dinv = _dinv_of(d_ref[...])
        hp1_ref[...] = g1_ref[...] * dinv
        hp2_ref[...] = g2_ref[...] * dinv

    return pl.pallas_call(
        body,
        grid=(n // BLK,),
        in_specs=[
            pl.BlockSpec((BLK, H), lambda i: (i, 0)),
            pl.BlockSpec((BLK, H), lambda i: (i, 0)),
            pl.BlockSpec((BLK, DW), lambda i: (i, 0)),
        ],
        out_specs=[
            pl.BlockSpec((BLK, H), lambda i: (i, 0)),
            pl.BlockSpec((BLK, H), lambda i: (i, 0)),
        ],
        out_shape=[
            jax.ShapeDtypeStruct((n, H), jnp.float32),
            jax.ShapeDtypeStruct((n, H), jnp.float32),
        ],
    )(g1, g2, degacc)


def _tc_mid(acc2, hp2, degacc, W3, b2):
    n = acc2.shape[0]

    def body(a_ref, hp_ref, d_ref, w3_ref, b2_ref, out_ref):
        dinv = _dinv_of(d_ref[...])
        h2 = _elu(dinv * (a_ref[...] + hp_ref[...]) + b2_ref[...])
        out_ref[...] = jnp.dot(h2, w3_ref[...],
                               preferred_element_type=jnp.float32) * dinv

    return pl.pallas_call(
        body,
        grid=(n // BLK,),
        in_specs=[
            pl.BlockSpec((BLK, H), lambda i: (i, 0)),
            pl.BlockSpec((BLK, H), lambda i: (i, 0)),
            pl.BlockSpec((BLK, DW), lambda i: (i, 0)),
            pl.BlockSpec((H, H), lambda i: (0, 0)),
            pl.BlockSpec((1, H), lambda i: (0, 0)),
        ],
        out_specs=pl.BlockSpec((BLK, H), lambda i: (i, 0)),
        out_shape=jax.ShapeDtypeStruct((n, H), jnp.float32),
    )(acc2, hp2, degacc, W3, b2)


def _tc_x1(acc1, hp1, degacc, b1):
    n = acc1.shape[0]

    def body(a_ref, hp_ref, d_ref, b_ref, out_ref):
        dinv = _dinv_of(d_ref[...])
        out_ref[...] = _elu(dinv * (a_ref[...] + hp_ref[...]) + b_ref[...])

    return pl.pallas_call(
        body,
        grid=(n // BLK,),
        in_specs=[
            pl.BlockSpec((BLK, H), lambda i: (i, 0)),
            pl.BlockSpec((BLK, H), lambda i: (i, 0)),
            pl.BlockSpec((BLK, DW), lambda i: (i, 0)),
            pl.BlockSpec((1, H), lambda i: (0, 0)),
        ],
        out_specs=pl.BlockSpec((BLK, H), lambda i: (i, 0)),
        out_shape=jax.ShapeDtypeStruct((n, H), jnp.float32),
    )(acc1, hp1, degacc, b1)


def _tc_fin(x1, acc3, hp3, b3, degacc, Wl, bl):
    n = x1.shape[0]
    c_out = Wl.shape[1]

    def body(x1_ref, a3_ref, hp3_ref, b3_ref, d_ref, wl_ref, bl_ref, out_ref):
        dinv = _dinv_of(d_ref[...])
        h3 = _elu(dinv * (a3_ref[...] + hp3_ref[...]) + b3_ref[...])
        h = x1_ref[...] + h3
        logits = jnp.dot(h, wl_ref[...],
                         preferred_element_type=jnp.float32) + bl_ref[...]
        m = jnp.max(logits, axis=-1, keepdims=True)
        lse = m + jnp.log(jnp.sum(jnp.exp(logits - m), axis=-1, keepdims=True))
        out_ref[...] = logits - lse

    return pl.pallas_call(
        body,
        grid=(n // BLK,),
        in_specs=[
            pl.BlockSpec((BLK, H), lambda i: (i, 0)),
            pl.BlockSpec((BLK, H), lambda i: (i, 0)),
            pl.BlockSpec((BLK, H), lambda i: (i, 0)),
            pl.BlockSpec((1, H), lambda i: (0, 0)),
            pl.BlockSpec((BLK, DW), lambda i: (i, 0)),
            pl.BlockSpec((H, c_out), lambda i: (0, 0)),
            pl.BlockSpec((1, c_out), lambda i: (0, 0)),
        ],
        out_specs=pl.BlockSpec((BLK, c_out), lambda i: (i, 0)),
        out_shape=jax.ShapeDtypeStruct((n, c_out), jnp.float32),
    )(x1, acc3, hp3, b3, degacc, Wl, bl)


def kernel(x, edge_index, edge_weight, W_conv1, b_conv1, W_conv2, b_conv2,
           W_conv3, b_conv3, W_first, b_first, W_lin2, b_lin2):
    n = x.shape[0]
    e = edge_index.shape[1]
    half = n // NC

    ep = _ceil_to(e, IB * NC * NS)
    pad = ep - e
    row = jnp.pad(edge_index[0], (0, pad))
    col = jnp.pad(edge_index[1], (0, pad))
    ew = jnp.pad(edge_weight, (0, pad))

    tot = _ceil_to(half + NS * 8, NS * 64)

    b1 = b_conv1.reshape(1, H)
    b2 = b_conv2.reshape(1, H)
    b3 = b_conv3.reshape(1, H)
    bf = b_first.reshape(1, H)
    bl = b_lin2.reshape(1, W_lin2.shape[1])

    ep32 = ep // (NC * NS)
    cap = _ceil_to(ep32, SEGQ) + SEGQ

    g1, g2 = _tc_mm(x, W_conv1, W_first, bf, W_conv2)
    rowp, colp, ewp, cnts = _partition(row, col, ew, half, cap)
    degacc = _deg_sum(col, ew, n, tot, half)
    hp1, hp2 = _tc_scale(g1, g2, degacc)
    acc1 = _seg_sum(hp1, rowp, colp, ewp, cnts, n, tot, half, cap)
    acc2 = _seg_sum(hp2, rowp, colp, ewp, cnts, n, tot, half, cap)
    x1 = _tc_x1(acc1, hp1, degacc, b1)
    hp3 = _tc_mid(acc2, hp2, degacc, W_conv3, b2)
    acc3 = _seg_sum(hp3, rowp, colp, ewp, cnts, n, tot, half, cap)
    return _tc_fin(x1, acc3, hp3, b3, degacc, W_lin2, bl)


# speedup vs baseline: 1.0020x; 1.0020x over previous
import functools

import jax
import jax.numpy as jnp
from jax import lax
from jax.experimental import pallas as pl
from jax.experimental.pallas import tpu as pltpu
from jax.experimental.pallas import tpu_sc as plsc

NC = 2
NS = 16
LN = 16

H = 32
DW = 16

CHUNK = 128
RPB = 8


def _ceil_to(x, m):
    return (x + m - 1) // m * m


IB = RPB * CHUNK
SEGQ = 2 * IB


def _partition(row1d, col1d, ew1d, half, cap):
    ep = row1d.shape[0]
    ep32 = ep // (NC * NS)
    nblk = ep32 // IB

    mesh = plsc.VectorSubcoreMesh(core_axis_name="c", subcore_axis_name="s",
                                  num_cores=NC, num_subcores=NS)
    nreg = NC * NS

    @functools.partial(
        pl.kernel,
        out_type=[
            jax.ShapeDtypeStruct((2 * nreg * cap,), jnp.int32),
            jax.ShapeDtypeStruct((2 * nreg * cap,), jnp.int32),
            jax.ShapeDtypeStruct((2 * nreg * cap,), jnp.float32),
            jax.ShapeDtypeStruct((2 * nreg * 16,), jnp.int32),
        ],
        mesh=mesh,
        compiler_params=pltpu.CompilerParams(use_tc_tiling_on_sc=False,
                                             needs_layout_passes=False),
        scratch_types=[
            pltpu.VMEM((IB,), jnp.int32),
            pltpu.VMEM((IB,), jnp.int32),
            pltpu.VMEM((IB,), jnp.float32),
            pltpu.VMEM((160,), jnp.int32),
            pltpu.VMEM((160,), jnp.int32),
            pltpu.VMEM((160,), jnp.float32),
            pltpu.VMEM((160,), jnp.int32),
            pltpu.VMEM((160,), jnp.int32),
            pltpu.VMEM((160,), jnp.float32),
            pltpu.VMEM((CHUNK,), jnp.int32),
            pltpu.VMEM((CHUNK,), jnp.float32),
            pltpu.VMEM((16,), jnp.int32),
        ],
    )
    def kern(row_hbm, col_hbm, ew_hbm, rowp, colp, ewp, cnts,
             rowv, colv, ewv, sr0, sc0, se0, sr1, sc1, se1, zi, zf, cntv):
        cid = lax.axis_index("c")
        sid = lax.axis_index("s")
        w = sid * NC + cid
        in_base = w * ep32
        stg = ((sr0, sc0, se0), (sr1, sc1, se1))
        outs = (rowp, colp, ewp)

        @pl.loop(0, CHUNK, step=LN)
        def _(r):
            zi[pl.ds(r, LN)] = jnp.zeros((LN,), jnp.int32)
            zf[pl.ds(r, LN)] = jnp.zeros((LN,), jnp.float32)

        def flush(p, optr):
            ob = pl.multiple_of((p * nreg + w) * cap + optr, CHUNK)
            for st, out in zip(stg[p], outs):
                pltpu.sync_copy(st.at[pl.ds(0, CHUNK)],
                                out.at[pl.ds(ob, CHUNK)])

        def shift_residual(p):
            for st in stg[p]:
                res = st[pl.ds(CHUNK, LN)]
                st[pl.ds(0, LN)] = res

        def compact_group(vals, m, p, sptr, optr):
            cnt = plsc.all_reduce_population_count(m)[0]
            for st, v in zip(stg[p], vals):
                plsc.store_compressed(st.at[pl.ds(sptr, LN)], v, mask=m)
            sptr = sptr + cnt

            def do_flush(args):
                sptr, optr = args
                flush(p, optr)
                shift_residual(p)
                return sptr - CHUNK, optr + CHUNK

            return lax.cond(sptr >= CHUNK, do_flush, lambda a: a, (sptr, optr))

        def block_body(b, carry):
            s0, o0, s1, o1 = carry
            e0 = pl.multiple_of(in_base + b * IB, CHUNK)
            pltpu.sync_copy(row_hbm.at[pl.ds(e0, IB)], rowv)
            pltpu.sync_copy(col_hbm.at[pl.ds(e0, IB)], colv)
            pltpu.sync_copy(ew_hbm.at[pl.ds(e0, IB)], ewv)

            def group_body(q, carry):
                s0, o0, s1, o1 = carry
                r16 = rowv[pl.ds(q * LN, LN)]
                c16 = colv[pl.ds(q * LN, LN)]
                e16 = ewv[pl.ds(q * LN, LN)]
                m1 = c16 >= half
                s0, o0 = compact_group((r16, c16, e16), ~m1, 0, s0, o0)
                s1, o1 = compact_group((r16, c16 - half, e16), m1, 1, s1, o1)
                return s0, o0, s1, o1

            return lax.fori_loop(0, IB // LN, group_body, (s0, o0, s1, o1))

        s0, o0, s1, o1 = lax.fori_loop(0, nblk, block_body, (0, 0, 0, 0))

        def finish(p, sptr, optr):
            rpad = (LN - sptr % LN) % LN
            m = lax.iota(jnp.int32, LN) < rpad
            for st in stg[p]:
                plsc.store_compressed(st.at[pl.ds(sptr, LN)],
                                      jnp.zeros((LN,), st.dtype), mask=m)
            sptr = sptr + rpad

            d16 = ((CHUNK - sptr % CHUNK) % CHUNK) // LN

            def pad_body(i, sp):
                for st in stg[p]:
                    st[pl.ds(sp, LN)] = jnp.zeros((LN,), st.dtype)
                return sp + LN

            sptr = lax.fori_loop(0, d16, pad_body, sptr)

            @pl.when(sptr > 0)
            def _():
                flush(p, optr)

            optr = optr + sptr
            target = jnp.maximum((optr + SEGQ - 1) // SEGQ * SEGQ, SEGQ)
            kd = (target - optr) // CHUNK

            def dummy_body(i, o):
                ob = pl.multiple_of((p * nreg + w) * cap + o, CHUNK)
                pltpu.sync_copy(zi, rowp.at[pl.ds(ob, CHUNK)])
                pltpu.sync_copy(zi, colp.at[pl.ds(ob, CHUNK)])
                pltpu.sync_copy(zf, ewp.at[pl.ds(ob, CHUNK)])
                return o + CHUNK

            lax.fori_loop(0, kd, dummy_body, optr)
            cntv[...] = jnp.full((LN,), target, jnp.int32)
            cb = pl.multiple_of((p * nreg + w) * 16, 16)
            pltpu.sync_copy(cntv, cnts.at[pl.ds(cb, 16)])

        finish(0, s0, o0)
        finish(1, s1, o1)

    return kern(row1d, col1d, ew1d)


def _seg_sum(hp, rowp, colp, ewp, counts, n, tot, half, cap):
    wb_a = _ceil_to(-(-half // NS), 8)
    wb_last = half - (NS - 1) * wb_a
    nreg = NC * NS

    mesh = plsc.VectorSubcoreMesh(core_axis_name="c", subcore_axis_name="s",
                                  num_cores=NC, num_subcores=NS)

    @functools.partial(
        pl.kernel,
        out_type=jax.ShapeDtypeStruct((n, H), jnp.float32),
        mesh=mesh,
        compiler_params=pltpu.CompilerParams(use_tc_tiling_on_sc=False),
        scratch_types=[
            pltpu.VMEM((2, RPB * CHUNK), jnp.int32),
            pltpu.VMEM((2, RPB * CHUNK), jnp.int32),
            pltpu.VMEM((2, RPB * CHUNK), jnp.float32),
            pltpu.VMEM((2, CHUNK), jnp.int32),
            pltpu.VMEM((2, CHUNK, H), jnp.float32),
            pltpu.VMEM((2, CHUNK, H), jnp.float32),
            pltpu.VMEM((64, H), jnp.float32),
            pltpu.VMEM((16,), jnp.int32),
            pltpu.VMEM_SHARED((tot, H), jnp.float32),
            pltpu.SemaphoreType.DMA,
            pltpu.SemaphoreType.DMA,
            pltpu.SemaphoreType.DMA,
        ],
    )
    def kern(hp_hbm, row_hbm, col_hbm, ew_hbm, cnt_hbm, out_hbm,
             rowv, colv, ewv, lidxv, gathv, msgv, zv, cntv, accs,
             sem_i, sem_g, sem_s):
        cid = lax.axis_index("c")
        sid = lax.axis_index("s")
        base = cid * half
        ib = RPB * CHUNK

        @pl.loop(0, 64)
        def _(r):
            @pl.loop(0, H, step=LN)
            def _(hh):
                zv.at[r][pl.ds(hh, LN)] = jnp.zeros((LN,), jnp.float32)

        zslice = tot // NS

        @pl.loop(0, zslice, step=64)
        def _(r):
            pltpu.sync_copy(zv, accs.at[pl.ds(sid * zslice + r, 64)])

        plsc.subcore_barrier()

        def idx_issue(ebase, b, pi):
            e0 = pl.multiple_of(ebase + b * ib, CHUNK)
            pltpu.async_copy(row_hbm.at[pl.ds(e0, ib)], rowv.at[pi], sem_i)
            pltpu.async_copy(col_hbm.at[pl.ds(e0, ib)], colv.at[pi], sem_i)
            pltpu.async_copy(ew_hbm.at[pl.ds(e0, ib)], ewv.at[pi], sem_i)

        def idx_drain(pi):
            pltpu.make_async_copy(row_hbm.at[pl.ds(0, ib)], rowv.at[pi], sem_i).wait()
            pltpu.make_async_copy(col_hbm.at[pl.ds(0, ib)], colv.at[pi], sem_i).wait()
            pltpu.make_async_copy(ew_hbm.at[pl.ds(0, ib)], ewv.at[pi], sem_i).wait()

        def gat_issue(c, pi):
            pltpu.async_copy(
                hp_hbm.at[rowv.at[pi].at[pl.ds(c * CHUNK, CHUNK)]],
                gathv.at[c % 2], sem_g)

        def gat_drain(c, pi):
            pltpu.make_async_copy(
                hp_hbm.at[rowv.at[pi].at[pl.ds(c * CHUNK, CHUNK)]],
                gathv.at[c % 2], sem_g).wait()

        def sc_issue(sp):
            pltpu.async_copy(msgv.at[sp], accs.at[lidxv.at[sp]], sem_s, add=True)

        def sc_drain(sp):
            pltpu.make_async_copy(msgv.at[sp], accs.at[lidxv.at[sp]], sem_s).wait()

        def do_chunk(bb, c, sp, pi):
            gat_drain(c, pi)

            @pl.when((bb > 0) | (c >= 2))
            def _():
                sc_drain(sp)

            for q in range(CHUNK // LN):
                lidxv.at[sp][pl.ds(q * LN, LN)] = (
                    colv.at[pi][pl.ds(c * CHUNK + q * LN, LN)])
                ew16 = ewv.at[pi][pl.ds(c * CHUNK + q * LN, LN)]
                for j in range(LN):
                    e = q * LN + j
                    w = jnp.full((LN,), ew16[j], jnp.float32)
                    mrow = msgv.at[sp].at[e]
                    grow = gathv.at[sp].at[e]
                    for hh in range(H // LN):
                        sl = pl.ds(hh * LN, LN)
                        mrow[sl] = grow[sl] * w

            sc_issue(sp)

        def do_block(ebase, nblk_r, bb, pi):
            idx_drain(pi)

            @pl.when(bb + 1 < nblk_r)
            def _():
                idx_issue(ebase, bb + 1, 1 - pi)

            gat_issue(0, pi)

            @pl.loop(0, RPB, step=2)
            def _(c):
                for off in (0, 1):
                    cc = c + off

                    @pl.when(cc + 1 < RPB)
                    def _():
                        gat_issue(cc + 1, pi)

                    do_chunk(bb, cc, off, pi)

        @pl.loop(0, 2)
        def _(k):
            reg = 2 * sid + k
            slot = cid * nreg + reg
            ebase = slot * cap
            pltpu.sync_copy(cnt_hbm.at[pl.ds(pl.multiple_of(slot * 16, 16), 16)],
                            cntv)
            nblk_r = cntv[...][0] // ib

            idx_issue(ebase, 0, 0)

            @pl.loop(0, nblk_r, step=2)
            def _(b):
                do_block(ebase, nblk_r, b, 0)
                do_block(ebase, nblk_r, b + 1, 1)

            sc_drain(0)
            sc_drain(1)

        plsc.subcore_barrier()

        @pl.when(sid < NS - 1)
        def _():
            pltpu.sync_copy(accs.at[pl.ds(sid * wb_a, wb_a)],
                            out_hbm.at[pl.ds(base + sid * wb_a, wb_a)])

        @pl.when(sid == NS - 1)
        def _():
            pltpu.sync_copy(accs.at[pl.ds((NS - 1) * wb_a, wb_last)],
                            out_hbm.at[pl.ds(base + (NS - 1) * wb_a, wb_last)])

    return kern(hp, rowp, colp, ewp, counts)


def _deg_sum(col1d, ew1d, n, tot, half):
    ep = col1d.shape[0]
    per_sub = ep // NS
    nblk = per_sub // (RPB * CHUNK)
    wb_a = _ceil_to(-(-half // NS), 8)
    wb_last = half - (NS - 1) * wb_a

    mesh = plsc.VectorSubcoreMesh(core_axis_name="c", subcore_axis_name="s",
                                  num_cores=NC, num_subcores=NS)

    @functools.partial(
        pl.kernel,
        out_type=jax.ShapeDtypeStruct((n, DW), jnp.float32),
        mesh=mesh,
        compiler_params=pltpu.CompilerParams(use_tc_tiling_on_sc=False),
        scratch_types=[
            pltpu.VMEM((2, RPB * CHUNK), jnp.int32),
            pltpu.VMEM((2, RPB * CHUNK), jnp.float32),
            pltpu.VMEM((2, CHUNK), jnp.int32),
            pltpu.VMEM((2, CHUNK, DW), jnp.float32),
            pltpu.VMEM((64, DW), jnp.float32),
            pltpu.VMEM_SHARED((tot, DW), jnp.float32),
            pltpu.SemaphoreType.DMA,
            pltpu.SemaphoreType.DMA,
        ],
    )
    def kern(col_hbm, ew_hbm, out_hbm, colv, ewv, lidxv, msgv, zv, accs,
             sem_i, sem_s):
        cid = lax.axis_index("c")
        sid = lax.axis_index("s")
        base = cid * half
        ib = RPB * CHUNK

        @pl.loop(0, 64)
        def _(r):
            zv.at[r][...] = jnp.zeros((LN,), jnp.float32)

        zslice = tot // NS

        @pl.loop(0, zslice, step=64)
        def _(r):
            pltpu.sync_copy(zv, accs.at[pl.ds(sid * zslice + r, 64)])

        plsc.subcore_barrier()

        def idx_issue(b, pi):
            e0 = sid * per_sub + b * ib
            pltpu.async_copy(col_hbm.at[pl.ds(e0, ib)], colv.at[pi], sem_i)
            pltpu.async_copy(ew_hbm.at[pl.ds(e0, ib)], ewv.at[pi], sem_i)

        def idx_drain(pi):
            pltpu.make_async_copy(col_hbm.at[pl.ds(0, ib)], colv.at[pi], sem_i).wait()
            pltpu.make_async_copy(ew_hbm.at[pl.ds(0, ib)], ewv.at[pi], sem_i).wait()

        def sc_issue(sp):
            pltpu.async_copy(msgv.at[sp], accs.at[lidxv.at[sp]], sem_s, add=True)

        def sc_drain(sp):
            pltpu.make_async_copy(msgv.at[sp], accs.at[lidxv.at[sp]], sem_s).wait()

        def do_chunk(bb, c, sp, pi):
            @pl.when((bb > 0) | (c >= 2))
            def _():
                sc_drain(sp)

            for q in range(CHUNK // LN):
                cols = colv.at[pi][pl.ds(c * CHUNK + q * LN, LN)]
                loc = cols - base
                oob = (loc < 0) | (loc >= half)
                trash = half + sid * 8 + (q % 8)
                loc = jnp.where(oob, jnp.full((LN,), trash, jnp.int32), loc)
                lidxv.at[sp][pl.ds(q * LN, LN)] = loc
                ew16 = ewv.at[pi][pl.ds(c * CHUNK + q * LN, LN)]
                for j in range(LN):
                    e = q * LN + j
                    msgv.at[sp].at[e][...] = jnp.full((LN,), ew16[j], jnp.float32)

            sc_issue(sp)

        def do_block(bb, pi):
            idx_drain(pi)

            @pl.when(bb + 1 < nblk)
            def _():
                idx_issue(bb + 1, 1 - pi)

            @pl.loop(0, RPB, step=2)
            def _(c):
                for off in (0, 1):
                    do_chunk(bb, c + off, off, pi)

        idx_issue(0, 0)

        @pl.loop(0, nblk, step=2)
        def _(b):
            do_block(b, 0)
            do_block(b + 1, 1)

        sc_drain(0)
        sc_drain(1)

        plsc.subcore_barrier()

        @pl.when(sid < NS - 1)
        def _():
            pltpu.sync_copy(accs.at[pl.ds(sid * wb_a, wb_a)],
                            out_hbm.at[pl.ds(base + sid * wb_a, wb_a)])

        @pl.when(sid == NS - 1)
        def _():
            pltpu.sync_copy(accs.at[pl.ds((NS - 1) * wb_a, wb_last)],
                            out_hbm.at[pl.ds(base + (NS - 1) * wb_a, wb_last)])

    return kern(col1d, ew1d)


BLK = 2000


def _elu(x):
    return jnp.where(x > 0, x, jnp.exp(x) - 1.0)


def _dinv_of(degacc):
    deg = degacc[:, :1] + 1.0
    return jnp.where(deg > 0, lax.rsqrt(jnp.where(deg > 0, deg, 1.0)), 0.0)


def _tc_mm(x, W1, Wf, bf, W2):
    n, f_in = x.shape

    def body(x_ref, w1_ref, wf_ref, bf_ref, w2_ref, g1_ref, g2_ref):
        xb = x_ref[...]
        g1_ref[...] = jnp.dot(xb, w1_ref[...],
                              preferred_element_type=jnp.float32)
        h0 = _elu(jnp.dot(xb, wf_ref[...],
                          preferred_element_type=jnp.float32) + bf_ref[...])
        g2_ref[...] = jnp.dot(h0, w2_ref[...],
                              preferred_element_type=jnp.float32)

    return pl.pallas_call(
        body,
        grid=(n // BLK,),
        in_specs=[
            pl.BlockSpec((BLK, f_in), lambda i: (i, 0)),
            pl.BlockSpec((f_in, H), lambda i: (0, 0)),
            pl.BlockSpec((f_in, H), lambda i: (0, 0)),
            pl.BlockSpec((1, H), lambda i: (0, 0)),
            pl.BlockSpec((H, H), lambda i: (0, 0)),
        ],
        out_specs=[
            pl.BlockSpec((BLK, H), lambda i: (i, 0)),
            pl.BlockSpec((BLK, H), lambda i: (i, 0)),
        ],
        out_shape=[
            jax.ShapeDtypeStruct((n, H), jnp.float32),
            jax.ShapeDtypeStruct((n, H), jnp.float32),
        ],
    )(x, W1, Wf, bf, W2)


def _tc_scale(g1, g2, degacc):
    n = g1.shape[0]

    def body(g1_ref, g2_ref, d_ref, hp1_ref, hp2_ref):
        dinv = _dinv_of(d_ref[...])
        hp1_ref[...] = g1_ref[...] * dinv
        hp2_ref[...] = g2_ref[...] * dinv

    return pl.pallas_call(
        body,
        grid=(n // BLK,),
        in_specs=[
            pl.BlockSpec((BLK, H), lambda i: (i, 0)),
            pl.BlockSpec((BLK, H), lambda i: (i, 0)),
            pl.BlockSpec((BLK, DW), lambda i: (i, 0)),
        ],
        out_specs=[
            pl.BlockSpec((BLK, H), lambda i: (i, 0)),
            pl.BlockSpec((BLK, H), lambda i: (i, 0)),
        ],
        out_shape=[
            jax.ShapeDtypeStruct((n, H), jnp.float32),
            jax.ShapeDtypeStruct((n, H), jnp.float32),
        ],
    )(g1, g2, degacc)


def _tc_mid(acc2, hp2, degacc, W3, b2):
    n = acc2.shape[0]

    def body(a_ref, hp_ref, d_ref, w3_ref, b2_ref, out_ref):
        dinv = _dinv_of(d_ref[...])
        h2 = _elu(dinv * (a_ref[...] + hp_ref[...]) + b2_ref[...])
        out_ref[...] = jnp.dot(h2, w3_ref[...],
                               preferred_element_type=jnp.float32) * dinv

    return pl.pallas_call(
        body,
        grid=(n // BLK,),
        in_specs=[
            pl.BlockSpec((BLK, H), lambda i: (i, 0)),
            pl.BlockSpec((BLK, H), lambda i: (i, 0)),
            pl.BlockSpec((BLK, DW), lambda i: (i, 0)),
            pl.BlockSpec((H, H), lambda i: (0, 0)),
            pl.BlockSpec((1, H), lambda i: (0, 0)),
        ],
        out_specs=pl.BlockSpec((BLK, H), lambda i: (i, 0)),
        out_shape=jax.ShapeDtypeStruct((n, H), jnp.float32),
    )(acc2, hp2, degacc, W3, b2)


def _tc_x1(acc1, hp1, degacc, b1):
    n = acc1.shape[0]

    def body(a_ref, hp_ref, d_ref, b_ref, out_ref):
        dinv = _dinv_of(d_ref[...])
        out_ref[...] = _elu(dinv * (a_ref[...] + hp_ref[...]) + b_ref[...])

    return pl.pallas_call(
        body,
        grid=(n // BLK,),
        in_specs=[
            pl.BlockSpec((BLK, H), lambda i: (i, 0)),
            pl.BlockSpec((BLK, H), lambda i: (i, 0)),
            pl.BlockSpec((BLK, DW), lambda i: (i, 0)),
            pl.BlockSpec((1, H), lambda i: (0, 0)),
        ],
        out_specs=pl.BlockSpec((BLK, H), lambda i: (i, 0)),
        out_shape=jax.ShapeDtypeStruct((n, H), jnp.float32),
    )(acc1, hp1, degacc, b1)


def _tc_fin(x1, acc3, hp3, b3, degacc, Wl, bl):
    n = x1.shape[0]
    c_out = Wl.shape[1]

    def body(x1_ref, a3_ref, hp3_ref, b3_ref, d_ref, wl_ref, bl_ref, out_ref):
        dinv = _dinv_of(d_ref[...])
        h3 = _elu(dinv * (a3_ref[...] + hp3_ref[...]) + b3_ref[...])
        h = x1_ref[...] + h3
        logits = jnp.dot(h, wl_ref[...],
                         preferred_element_type=jnp.float32) + bl_ref[...]
        m = jnp.max(logits, axis=-1, keepdims=True)
        lse = m + jnp.log(jnp.sum(jnp.exp(logits - m), axis=-1, keepdims=True))
        out_ref[...] = logits - lse

    return pl.pallas_call(
        body,
        grid=(n // BLK,),
        in_specs=[
            pl.BlockSpec((BLK, H), lambda i: (i, 0)),
            pl.BlockSpec((BLK, H), lambda i: (i, 0)),
            pl.BlockSpec((BLK, H), lambda i: (i, 0)),
            pl.BlockSpec((1, H), lambda i: (0, 0)),
            pl.BlockSpec((BLK, DW), lambda i: (i, 0)),
            pl.BlockSpec((H, c_out), lambda i: (0, 0)),
            pl.BlockSpec((1, c_out), lambda i: (0, 0)),
        ],
        out_specs=pl.BlockSpec((BLK, c_out), lambda i: (i, 0)),
        out_shape=jax.ShapeDtypeStruct((n, c_out), jnp.float32),
    )(x1, acc3, hp3, b3, degacc, Wl, bl)


def kernel(x, edge_index, edge_weight, W_conv1, b_conv1, W_conv2, b_conv2,
           W_conv3, b_conv3, W_first, b_first, W_lin2, b_lin2):
    n = x.shape[0]
    e = edge_index.shape[1]
    half = n // NC

    ep = _ceil_to(e, IB * NC * NS)
    pad = ep - e
    row = jnp.pad(edge_index[0], (0, pad))
    col = jnp.pad(edge_index[1], (0, pad))
    ew = jnp.pad(edge_weight, (0, pad))

    tot = _ceil_to(half + NS * 8, NS * 64)

    b1 = b_conv1.reshape(1, H)
    b2 = b_conv2.reshape(1, H)
    b3 = b_conv3.reshape(1, H)
    bf = b_first.reshape(1, H)
    bl = b_lin2.reshape(1, W_lin2.shape[1])

    ep32 = ep // (NC * NS)
    cap = _ceil_to(ep32, SEGQ) + SEGQ

    g1, g2 = _tc_mm(x, W_conv1, W_first, bf, W_conv2)
    rowp, colp, ewp, cnts = _partition(row, col, ew, half, cap)
    degacc = _deg_sum(col, ew, n, tot, half)
    hp1, hp2 = _tc_scale(g1, g2, degacc)
    acc1 = _seg_sum(hp1, rowp, colp, ewp, cnts, n, tot, half, cap)
    acc2 = _seg_sum(hp2, rowp, colp, ewp, cnts, n, tot, half, cap)
    x1 = _tc_x1(acc1, hp1, degacc, b1)
    hp3 = _tc_mid(acc2, hp2, degacc, W_conv3, b2)
    acc3 = _seg_sum(hp3, rowp, colp, ewp, cnts, n, tot, half, cap)
    return _tc_fin(x1, acc3, hp3, b3, degacc, W_lin2, bl)


# speedup vs baseline: 1.9477x; 1.9438x over previous
import functools

import jax
import jax.numpy as jnp
from jax import lax
from jax.experimental import pallas as pl
from jax.experimental.pallas import tpu as pltpu
from jax.experimental.pallas import tpu_sc as plsc

NC = 2
NS = 16
LN = 16

H = 32
DW = 16

CHUNK = 128
RPB = 8


def _ceil_to(x, m):
    return (x + m - 1) // m * m


def _seg_sum(hp, row1d, col1d, ew1d, n, tot, half):
    ep = row1d.shape[0]
    per_sub = ep // NS
    nblk = per_sub // (RPB * CHUNK)
    wb_a = _ceil_to(-(-half // NS), 8)
    wb_last = half - (NS - 1) * wb_a

    mesh = plsc.VectorSubcoreMesh(core_axis_name="c", subcore_axis_name="s",
                                  num_cores=NC, num_subcores=NS)

    @functools.partial(
        pl.kernel,
        out_type=jax.ShapeDtypeStruct((n, H), jnp.float32),
        mesh=mesh,
        compiler_params=pltpu.CompilerParams(use_tc_tiling_on_sc=False),
        scratch_types=[
            pltpu.VMEM((2, RPB * CHUNK), jnp.int32),
            pltpu.VMEM((2, RPB * CHUNK), jnp.int32),
            pltpu.VMEM((2, RPB * CHUNK), jnp.float32),
            pltpu.VMEM((2, CHUNK), jnp.int32),
            pltpu.VMEM((2, CHUNK, H), jnp.float32),
            pltpu.VMEM((2, CHUNK, H), jnp.float32),
            pltpu.VMEM((64, H), jnp.float32),
            pltpu.VMEM_SHARED((tot, H), jnp.float32),
            pltpu.SemaphoreType.DMA,
            pltpu.SemaphoreType.DMA,
            pltpu.SemaphoreType.DMA,
        ],
    )
    def kern(hp_hbm, row_hbm, col_hbm, ew_hbm, out_hbm,
             rowv, colv, ewv, lidxv, gathv, msgv, zv, accs,
             sem_i, sem_g, sem_s):
        cid = lax.axis_index("c")
        sid = lax.axis_index("s")
        base = cid * half
        ib = RPB * CHUNK

        @pl.loop(0, 64)
        def _(r):
            @pl.loop(0, H, step=LN)
            def _(hh):
                zv.at[r][pl.ds(hh, LN)] = jnp.zeros((LN,), jnp.float32)

        zslice = tot // NS

        @pl.loop(0, zslice, step=64)
        def _(r):
            pltpu.sync_copy(zv, accs.at[pl.ds(sid * zslice + r, 64)])

        plsc.subcore_barrier()

        def idx_issue(b, pi):
            e0 = sid * per_sub + b * ib
            pltpu.async_copy(row_hbm.at[pl.ds(e0, ib)], rowv.at[pi], sem_i)
            pltpu.async_copy(col_hbm.at[pl.ds(e0, ib)], colv.at[pi], sem_i)
            pltpu.async_copy(ew_hbm.at[pl.ds(e0, ib)], ewv.at[pi], sem_i)

        def idx_drain(pi):
            pltpu.make_async_copy(row_hbm.at[pl.ds(0, ib)], rowv.at[pi], sem_i).wait()
            pltpu.make_async_copy(col_hbm.at[pl.ds(0, ib)], colv.at[pi], sem_i).wait()
            pltpu.make_async_copy(ew_hbm.at[pl.ds(0, ib)], ewv.at[pi], sem_i).wait()

        def gat_issue(c, pi):
            pltpu.async_copy(
                hp_hbm.at[rowv.at[pi].at[pl.ds(c * CHUNK, CHUNK)]],
                gathv.at[c % 2], sem_g)

        def gat_drain(c, pi):
            pltpu.make_async_copy(
                hp_hbm.at[rowv.at[pi].at[pl.ds(c * CHUNK, CHUNK)]],
                gathv.at[c % 2], sem_g).wait()

        def sc_issue(sp):
            pltpu.async_copy(msgv.at[sp], accs.at[lidxv.at[sp]], sem_s, add=True)

        def sc_drain(sp):
            pltpu.make_async_copy(msgv.at[sp], accs.at[lidxv.at[sp]], sem_s).wait()

        def do_chunk(bb, c, sp, pi):
            gat_drain(c, pi)

            @pl.when((bb > 0) | (c >= 2))
            def _():
                sc_drain(sp)

            for q in range(CHUNK // LN):
                cols = colv.at[pi][pl.ds(c * CHUNK + q * LN, LN)]
                loc = cols - base
                oob = (loc < 0) | (loc >= half)
                trash = half + sid * 8 + (q % 8)
                loc = jnp.where(oob, jnp.full((LN,), trash, jnp.int32), loc)
                lidxv.at[sp][pl.ds(q * LN, LN)] = loc
                ew16 = ewv.at[pi][pl.ds(c * CHUNK + q * LN, LN)]
                for j in range(LN):
                    e = q * LN + j
                    w = jnp.full((LN,), ew16[j], jnp.float32)
                    mrow = msgv.at[sp].at[e]
                    grow = gathv.at[sp].at[e]
                    for hh in range(H // LN):
                        sl = pl.ds(hh * LN, LN)
                        mrow[sl] = grow[sl] * w

            sc_issue(sp)

        def do_block(bb, pi):
            idx_drain(pi)

            @pl.when(bb + 1 < nblk)
            def _():
                idx_issue(bb + 1, 1 - pi)

            gat_issue(0, pi)

            @pl.loop(0, RPB, step=2)
            def _(c):
                for off in (0, 1):
                    cc = c + off

                    @pl.when(cc + 1 < RPB)
                    def _():
                        gat_issue(cc + 1, pi)

                    do_chunk(bb, cc, off, pi)

        idx_issue(0, 0)

        @pl.loop(0, nblk, step=2)
        def _(b):
            do_block(b, 0)
            do_block(b + 1, 1)

        sc_drain(0)
        sc_drain(1)

        plsc.subcore_barrier()

        @pl.when(sid < NS - 1)
        def _():
            pltpu.sync_copy(accs.at[pl.ds(sid * wb_a, wb_a)],
                            out_hbm.at[pl.ds(base + sid * wb_a, wb_a)])

        @pl.when(sid == NS - 1)
        def _():
            pltpu.sync_copy(accs.at[pl.ds((NS - 1) * wb_a, wb_last)],
                            out_hbm.at[pl.ds(base + (NS - 1) * wb_a, wb_last)])

    return kern(hp, row1d, col1d, ew1d)


def _deg_sum(col1d, ew1d, n, tot, half):
    ep = col1d.shape[0]
    per_sub = ep // NS
    nblk = per_sub // (RPB * CHUNK)
    wb_a = _ceil_to(-(-half // NS), 8)
    wb_last = half - (NS - 1) * wb_a

    mesh = plsc.VectorSubcoreMesh(core_axis_name="c", subcore_axis_name="s",
                                  num_cores=NC, num_subcores=NS)

    @functools.partial(
        pl.kernel,
        out_type=jax.ShapeDtypeStruct((n, DW), jnp.float32),
        mesh=mesh,
        compiler_params=pltpu.CompilerParams(use_tc_tiling_on_sc=False),
        scratch_types=[
            pltpu.VMEM((2, RPB * CHUNK), jnp.int32),
            pltpu.VMEM((2, RPB * CHUNK), jnp.float32),
            pltpu.VMEM((2, CHUNK), jnp.int32),
            pltpu.VMEM((2, CHUNK, DW), jnp.float32),
            pltpu.VMEM((64, DW), jnp.float32),
            pltpu.VMEM_SHARED((tot, DW), jnp.float32),
            pltpu.SemaphoreType.DMA,
            pltpu.SemaphoreType.DMA,
        ],
    )
    def kern(col_hbm, ew_hbm, out_hbm, colv, ewv, lidxv, msgv, zv, accs,
             sem_i, sem_s):
        cid = lax.axis_index("c")
        sid = lax.axis_index("s")
        base = cid * half
        ib = RPB * CHUNK

        @pl.loop(0, 64)
        def _(r):
            zv.at[r][...] = jnp.zeros((LN,), jnp.float32)

        zslice = tot // NS

        @pl.loop(0, zslice, step=64)
        def _(r):
            pltpu.sync_copy(zv, accs.at[pl.ds(sid * zslice + r, 64)])

        plsc.subcore_barrier()

        def idx_issue(b, pi):
            e0 = sid * per_sub + b * ib
            pltpu.async_copy(col_hbm.at[pl.ds(e0, ib)], colv.at[pi], sem_i)
            pltpu.async_copy(ew_hbm.at[pl.ds(e0, ib)], ewv.at[pi], sem_i)

        def idx_drain(pi):
            pltpu.make_async_copy(col_hbm.at[pl.ds(0, ib)], colv.at[pi], sem_i).wait()
            pltpu.make_async_copy(ew_hbm.at[pl.ds(0, ib)], ewv.at[pi], sem_i).wait()

        def sc_issue(sp):
            pltpu.async_copy(msgv.at[sp], accs.at[lidxv.at[sp]], sem_s, add=True)

        def sc_drain(sp):
            pltpu.make_async_copy(msgv.at[sp], accs.at[lidxv.at[sp]], sem_s).wait()

        def do_chunk(bb, c, sp, pi):
            @pl.when((bb > 0) | (c >= 2))
            def _():
                sc_drain(sp)

            for q in range(CHUNK // LN):
                cols = colv.at[pi][pl.ds(c * CHUNK + q * LN, LN)]
                loc = cols - base
                oob = (loc < 0) | (loc >= half)
                trash = half + sid * 8 + (q % 8)
                loc = jnp.where(oob, jnp.full((LN,), trash, jnp.int32), loc)
                lidxv.at[sp][pl.ds(q * LN, LN)] = loc
                ew16 = ewv.at[pi][pl.ds(c * CHUNK + q * LN, LN)]
                for j in range(LN):
                    e = q * LN + j
                    msgv.at[sp].at[e][...] = jnp.full((LN,), ew16[j], jnp.float32)

            sc_issue(sp)

        def do_block(bb, pi):
            idx_drain(pi)

            @pl.when(bb + 1 < nblk)
            def _():
                idx_issue(bb + 1, 1 - pi)

            @pl.loop(0, RPB, step=2)
            def _(c):
                for off in (0, 1):
                    do_chunk(bb, c + off, off, pi)

        idx_issue(0, 0)

        @pl.loop(0, nblk, step=2)
        def _(b):
            do_block(b, 0)
            do_block(b + 1, 1)

        sc_drain(0)
        sc_drain(1)

        plsc.subcore_barrier()

        @pl.when(sid < NS - 1)
        def _():
            pltpu.sync_copy(accs.at[pl.ds(sid * wb_a, wb_a)],
                            out_hbm.at[pl.ds(base + sid * wb_a, wb_a)])

        @pl.when(sid == NS - 1)
        def _():
            pltpu.sync_copy(accs.at[pl.ds((NS - 1) * wb_a, wb_last)],
                            out_hbm.at[pl.ds(base + (NS - 1) * wb_a, wb_last)])

    return kern(col1d, ew1d)


BLK = 2000


def _elu(x):
    return jnp.where(x > 0, x, jnp.exp(x) - 1.0)


def _dinv_of(degacc):
    deg = degacc[:, :1] + 1.0
    return jnp.where(deg > 0, lax.rsqrt(jnp.where(deg > 0, deg, 1.0)), 0.0)


def _tc_mm(x, W1, Wf, bf, W2):
    n, f_in = x.shape

    def body(x_ref, w1_ref, wf_ref, bf_ref, w2_ref, g1_ref, g2_ref):
        xb = x_ref[...]
        g1_ref[...] = jnp.dot(xb, w1_ref[...],
                              preferred_element_type=jnp.float32)
        h0 = _elu(jnp.dot(xb, wf_ref[...],
                          preferred_element_type=jnp.float32) + bf_ref[...])
        g2_ref[...] = jnp.dot(h0, w2_ref[...],
                              preferred_element_type=jnp.float32)

    return pl.pallas_call(
        body,
        grid=(n // BLK,),
        in_specs=[
            pl.BlockSpec((BLK, f_in), lambda i: (i, 0)),
            pl.BlockSpec((f_in, H), lambda i: (0, 0)),
            pl.BlockSpec((f_in, H), lambda i: (0, 0)),
            pl.BlockSpec((1, H), lambda i: (0, 0)),
            pl.BlockSpec((H, H), lambda i: (0, 0)),
        ],
        out_specs=[
            pl.BlockSpec((BLK, H), lambda i: (i, 0)),
            pl.BlockSpec((BLK, H), lambda i: (i, 0)),
        ],
        out_shape=[
            jax.ShapeDtypeStruct((n, H), jnp.float32),
            jax.ShapeDtypeStruct((n, H), jnp.float32),
        ],
    )(x, W1, Wf, bf, W2)


def _tc_scale(g1, g2, degacc):
    n = g1.shape[0]

    def body(g1_ref, g2_ref, d_ref, hp1_ref, hp2_ref):
        dinv = _dinv_of(d_ref[...])
        hp1_ref[...] = g1_ref[...] * dinv
        hp2_ref[...] = g2_ref[...] * dinv

    return pl.pallas_call(
        body,
        grid=(n // BLK,),
        in_specs=[
            pl.BlockSpec((BLK, H), lambda i: (i, 0)),
            pl.BlockSpec((BLK, H), lambda i: (i, 0)),
            pl.BlockSpec((BLK, DW), lambda i: (i, 0)),
        ],
        out_specs=[
            pl.BlockSpec((BLK, H), lambda i: (i, 0)),
            pl.BlockSpec((BLK, H), lambda i: (i, 0)),
        ],
        out_shape=[
            jax.ShapeDtypeStruct((n, H), jnp.float32),
            jax.ShapeDtypeStruct((n, H), jnp.float32),
        ],
    )(g1, g2, degacc)


def _tc_mid(acc2, hp2, degacc, W3, b2):
    n = acc2.shape[0]

    def body(a_ref, hp_ref, d_ref, w3_ref, b2_ref, out_ref):
        dinv = _dinv_of(d_ref[...])
        h2 = _elu(dinv * (a_ref[...] + hp_ref[...]) + b2_ref[...])
        out_ref[...] = jnp.dot(h2, w3_ref[...],
                               preferred_element_type=jnp.float32) * dinv

    return pl.pallas_call(
        body,
        grid=(n // BLK,),
        in_specs=[
            pl.BlockSpec((BLK, H), lambda i: (i, 0)),
            pl.BlockSpec((BLK, H), lambda i: (i, 0)),
            pl.BlockSpec((BLK, DW), lambda i: (i, 0)),
            pl.BlockSpec((H, H), lambda i: (0, 0)),
            pl.BlockSpec((1, H), lambda i: (0, 0)),
        ],
        out_specs=pl.BlockSpec((BLK, H), lambda i: (i, 0)),
        out_shape=jax.ShapeDtypeStruct((n, H), jnp.float32),
    )(acc2, hp2, degacc, W3, b2)


def _tc_x1(acc1, hp1, degacc, b1):
    n = acc1.shape[0]

    def body(a_ref, hp_ref, d_ref, b_ref, out_ref):
        dinv = _dinv_of(d_ref[...])
        out_ref[...] = _elu(dinv * (a_ref[...] + hp_ref[...]) + b_ref[...])

    return pl.pallas_call(
        body,
        grid=(n // BLK,),
        in_specs=[
            pl.BlockSpec((BLK, H), lambda i: (i, 0)),
            pl.BlockSpec((BLK, H), lambda i: (i, 0)),
            pl.BlockSpec((BLK, DW), lambda i: (i, 0)),
            pl.BlockSpec((1, H), lambda i: (0, 0)),
        ],
        out_specs=pl.BlockSpec((BLK, H), lambda i: (i, 0)),
        out_shape=jax.ShapeDtypeStruct((n, H), jnp.float32),
    )(acc1, hp1, degacc, b1)


def _tc_fin(x1, acc3, hp3, b3, degacc, Wl, bl):
    n = x1.shape[0]
    c_out = Wl.shape[1]

    def body(x1_ref, a3_ref, hp3_ref, b3_ref, d_ref, wl_ref, bl_ref, out_ref):
        dinv = _dinv_of(d_ref[...])
        h3 = _elu(dinv * (a3_ref[...] + hp3_ref[...]) + b3_ref[...])
        h = x1_ref[...] + h3
        logits = jnp.dot(h, wl_ref[...],
                         preferred_element_type=jnp.float32) + bl_ref[...]
        m = jnp.max(logits, axis=-1, keepdims=True)
        lse = m + jnp.log(jnp.sum(jnp.exp(logits - m), axis=-1, keepdims=True))
        out_ref[...] = logits - lse

    return pl.pallas_call(
        body,
        grid=(n // BLK,),
        in_specs=[
            pl.BlockSpec((BLK, H), lambda i: (i, 0)),
            pl.BlockSpec((BLK, H), lambda i: (i, 0)),
            pl.BlockSpec((BLK, H), lambda i: (i, 0)),
            pl.BlockSpec((1, H), lambda i: (0, 0)),
            pl.BlockSpec((BLK, DW), lambda i: (i, 0)),
            pl.BlockSpec((H, c_out), lambda i: (0, 0)),
            pl.BlockSpec((1, c_out), lambda i: (0, 0)),
        ],
        out_specs=pl.BlockSpec((BLK, c_out), lambda i: (i, 0)),
        out_shape=jax.ShapeDtypeStruct((n, c_out), jnp.float32),
    )(x1, acc3, hp3, b3, degacc, Wl, bl)


def kernel(x, edge_index, edge_weight, W_conv1, b_conv1, W_conv2, b_conv2,
           W_conv3, b_conv3, W_first, b_first, W_lin2, b_lin2):
    n = x.shape[0]
    e = edge_index.shape[1]
    half = n // NC

    ep = _ceil_to(e, CHUNK * NS * RPB)
    pad = ep - e
    row = jnp.pad(edge_index[0], (0, pad))
    col = jnp.pad(edge_index[1], (0, pad))
    ew = jnp.pad(edge_weight, (0, pad))

    tot = _ceil_to(half + NS * 8, NS * 64)

    b1 = b_conv1.reshape(1, H)
    b2 = b_conv2.reshape(1, H)
    b3 = b_conv3.reshape(1, H)
    bf = b_first.reshape(1, H)
    bl = b_lin2.reshape(1, W_lin2.shape[1])

    g1, g2 = _tc_mm(x, W_conv1, W_first, bf, W_conv2)
    degacc = _deg_sum(col, ew, n, tot, half)
    hp1, hp2 = _tc_scale(g1, g2, degacc)
    acc1 = _seg_sum(hp1, row, col, ew, n, tot, half)
    acc2 = _seg_sum(hp2, row, col, ew, n, tot, half)
    x1 = _tc_x1(acc1, hp1, degacc, b1)
    hp3 = _tc_mid(acc2, hp2, degacc, W_conv3, b2)
    acc3 = _seg_sum(hp3, row, col, ew, n, tot, half)
    return _tc_fin(x1, acc3, hp3, b3, degacc, W_lin2, bl)
